# baseline (speedup 1.0000x reference)
import math

import jax
import jax.numpy as jnp
from jax import lax
from jax.experimental import pallas as pl
from jax.experimental.pallas import tpu as pltpu
from jax.experimental.pallas import tpu_sc as plsc

VOCAB = 1000000
D = 128
L = 200
B = 4096
N = B * L
NC, NS = 2, 16
NW = NC * NS
PER_W = N // NW
R = 128
S = PER_W // R
NBUF = 4
SQ = math.sqrt(float(D))


def _pe_table():
    pos = jnp.arange(0, L, dtype=jnp.float32)[:, None]
    div = jnp.exp(jnp.arange(0, D, 2, dtype=jnp.float32) * (-math.log(10000.0) / D))
    pe = jnp.zeros((L, D), dtype=jnp.float32)
    pe = pe.at[:, 0::2].set(jnp.sin(pos * div))
    pe = pe.at[:, 1::2].set(jnp.cos(pos * div))
    return pe


def _emb_body(idx_hbm, pe_hbm, table_hbm, out_hbm, idx_v, pe_v, rows, gsems, wsems):
    wid = lax.axis_index("s") * NC + lax.axis_index("c")
    base = wid * PER_W

    pltpu.sync_copy(idx_hbm.at[wid], idx_v)
    pltpu.sync_copy(pe_hbm, pe_v)

    def start_gather(step, b):
        return pltpu.async_copy(table_hbm.at[idx_v.at[step]], rows[b], gsems[b])

    def wait_gather(step, b):
        pltpu.make_async_copy(table_hbm.at[idx_v.at[step]], rows[b], gsems[b]).wait()

    def start_wb(step, b):
        return pltpu.async_copy(
            rows[b], out_hbm.at[pl.ds(base + step * R, R)], wsems[b])

    def wait_wb(step, b):
        pltpu.make_async_copy(
            rows[b], out_hbm.at[pl.ds(base + step * R, R)], wsems[b]).wait()

    def compute(b, s):
        buf = rows[b]
        pe_base = lax.rem(s * R, L)

        def row_body(r, carry):
            for rr in range(2):
                row = 2 * r + rr
                pr = pe_base + row
                pr = jnp.where(pr >= L, pr - L, pr)
                for c in range(D // 16):
                    sl = pl.ds(c * 16, 16)
                    buf[row, sl] = buf[row, sl] * SQ + pe_v[pr, sl]
            return carry

        lax.fori_loop(0, R // 2, row_body, 0)

    _SERIAL = False
    if _SERIAL:
        def body_serial(t, carry):
            for b in range(NBUF):
                s = t * NBUF + b
                start_gather(s, b)
                wait_gather(s, b)
                compute(b, s)
                start_wb(s, b)
                wait_wb(s, b)
            return carry
        lax.fori_loop(0, S // NBUF, body_serial, 0)
    else:
        for b in range(NBUF - 1):
            start_gather(b, b)

        def body(t, carry):
            for b in range(NBUF):
                s = t * NBUF + b
                wait_gather(s, b)
                compute(b, s)
                start_wb(s, b)
                g = s + NBUF - 1
                nb = (b + NBUF - 1) % NBUF
                if b == 0:
                    @pl.when(t >= 1)
                    def _():
                        wait_wb(s - 1, nb)
                    start_gather(g, nb)
                else:
                    @pl.when(g < S)
                    def _():
                        wait_wb(s - 1, nb)
                        start_gather(g, nb)
            return carry

        lax.fori_loop(0, S // NBUF, body, 0)

        for b in range(NBUF):
            wait_wb(S - NBUF + b, b)


def _emb_call(idx, pe, table):
    mesh = plsc.VectorSubcoreMesh(
        core_axis_name="c", subcore_axis_name="s", num_cores=NC, num_subcores=NS)
    return pl.kernel(
        _emb_body,
        out_type=jax.ShapeDtypeStruct((N, D), jnp.float32),
        mesh=mesh,
        scratch_types=[
            pltpu.VMEM((S, R), jnp.int32),
            pltpu.VMEM((L, D), jnp.float32),
            [pltpu.VMEM((R, D), jnp.float32) for _ in range(NBUF)],
            [pltpu.SemaphoreType.DMA for _ in range(NBUF)],
            [pltpu.SemaphoreType.DMA for _ in range(NBUF)],
        ],
    )(idx, pe, table)


def kernel(x, table):
    idx = x.reshape(NW, S, R)
    pe = _pe_table()
    out = _emb_call(idx, pe, table)
    return out.reshape(B, L, D)


# speedup vs baseline: 2.9773x; 2.9773x over previous
import math

import jax
import jax.numpy as jnp
from jax import lax
from jax.experimental import pallas as pl
from jax.experimental.pallas import tpu as pltpu
from jax.experimental.pallas import tpu_sc as plsc

VOCAB = 1000000
D = 128
L = 200
B = 4096
N = B * L
NC, NS = 2, 16
NW = NC * NS
PER_W = N // NW
R = 128
S = PER_W // R
NBUF = 4
SQ = math.sqrt(float(D))


def _pe_table():
    pos = jnp.arange(0, L, dtype=jnp.float32)[:, None]
    div = jnp.exp(jnp.arange(0, D, 2, dtype=jnp.float32) * (-math.log(10000.0) / D))
    pe = jnp.zeros((L, D), dtype=jnp.float32)
    pe = pe.at[:, 0::2].set(jnp.sin(pos * div))
    pe = pe.at[:, 1::2].set(jnp.cos(pos * div))
    return pe


def _emb_body(idx_hbm, pe_hbm, table_hbm, out_hbm, idx_v, pe_v, rows, gsems, wsems):
    wid = lax.axis_index("s") * NC + lax.axis_index("c")
    base = wid * PER_W

    pltpu.sync_copy(idx_hbm.at[wid], idx_v)
    pltpu.sync_copy(pe_hbm, pe_v)

    def start_gather(step, b):
        return pltpu.async_copy(table_hbm.at[idx_v.at[step]], rows[b], gsems[b])

    def wait_gather(step, b):
        pltpu.make_async_copy(table_hbm.at[idx_v.at[step]], rows[b], gsems[b]).wait()

    def start_wb(step, b):
        return pltpu.async_copy(
            rows[b], out_hbm.at[pl.ds(base + step * R, R)], wsems[b])

    def wait_wb(step, b):
        pltpu.make_async_copy(
            rows[b], out_hbm.at[pl.ds(base + step * R, R)], wsems[b]).wait()

    def compute(b, s):
        buf = rows[b]
        pe_base = lax.rem(s * R, L)

        @plsc.parallel_loop(0, R, unroll=4)
        def _row(row):
            pr = pe_base + row
            pr = jnp.where(pr >= L, pr - L, pr)
            for c in range(D // 16):
                sl = pl.ds(c * 16, 16)
                buf[row, sl] = buf[row, sl] * SQ + pe_v[pr, sl]

    _SERIAL = False
    if _SERIAL:
        def body_serial(t, carry):
            for b in range(NBUF):
                s = t * NBUF + b
                start_gather(s, b)
                wait_gather(s, b)
                compute(b, s)
                start_wb(s, b)
                wait_wb(s, b)
            return carry
        lax.fori_loop(0, S // NBUF, body_serial, 0)
    else:
        for b in range(NBUF - 1):
            start_gather(b, b)

        def body(t, carry):
            for b in range(NBUF):
                s = t * NBUF + b
                wait_gather(s, b)
                compute(b, s)
                start_wb(s, b)
                g = s + NBUF - 1
                nb = (b + NBUF - 1) % NBUF
                if b == 0:
                    @pl.when(t >= 1)
                    def _():
                        wait_wb(s - 1, nb)
                    start_gather(g, nb)
                else:
                    @pl.when(g < S)
                    def _():
                        wait_wb(s - 1, nb)
                        start_gather(g, nb)
            return carry

        lax.fori_loop(0, S // NBUF, body, 0)

        for b in range(NBUF):
            wait_wb(S - NBUF + b, b)


def _emb_call(idx, pe, table):
    mesh = plsc.VectorSubcoreMesh(
        core_axis_name="c", subcore_axis_name="s", num_cores=NC, num_subcores=NS)
    return pl.kernel(
        _emb_body,
        out_type=jax.ShapeDtypeStruct((N, D), jnp.float32),
        mesh=mesh,
        scratch_types=[
            pltpu.VMEM((S, R), jnp.int32),
            pltpu.VMEM((L, D), jnp.float32),
            [pltpu.VMEM((R, D), jnp.float32) for _ in range(NBUF)],
            [pltpu.SemaphoreType.DMA for _ in range(NBUF)],
            [pltpu.SemaphoreType.DMA for _ in range(NBUF)],
        ],
    )(idx, pe, table)


def kernel(x, table):
    idx = x.reshape(NW, S, R)
    pe = _pe_table()
    out = _emb_call(idx, pe, table)
    return out.reshape(B, L, D)
